# baseline (speedup 1.0000x reference)
import functools

import jax
import jax.numpy as jnp
from jax import lax
from jax.experimental import pallas as pl
from jax.experimental.pallas import tpu as pltpu
from jax.experimental.pallas import tpu_sc as plsc

E = 8
K = 2
D = 768
H = 1536
T = 4096
TK = T * K
BT = 256
PT = TK + E * BT
NB = PT // BT

GBT = 512

NC = 2
NS = 16
NW = NC * NS
NL = 16


def _gating_body(x_ref, Wg_ref, bg_ref, sel_ref, wgt_ref):
    logits = jnp.dot(x_ref[...], Wg_ref[...],
                     preferred_element_type=jnp.float32) + bg_ref[...]
    idx = jax.lax.broadcasted_iota(jnp.int32, logits.shape, 1)
    m1 = jnp.max(logits, axis=-1, keepdims=True)
    a1 = jnp.min(jnp.where(logits == m1, idx, E), axis=-1, keepdims=True)
    l2 = jnp.where(idx == a1, -jnp.inf, logits)
    m2 = jnp.max(l2, axis=-1, keepdims=True)
    a2 = jnp.min(jnp.where(l2 == m2, idx, E), axis=-1, keepdims=True)
    e2 = jnp.exp(m2 - m1)
    s = 1.0 + e2
    sel_ref[...] = jnp.concatenate([a1, a2], axis=1)
    wgt_ref[...] = jnp.concatenate([1.0 / s, e2 / s], axis=1)


def _gating(xf, Wg, bg):
    return pl.pallas_call(
        _gating_body,
        grid=(T // GBT,),
        in_specs=[
            pl.BlockSpec((GBT, D), lambda i: (i, 0)),
            pl.BlockSpec((D, E), lambda i: (0, 0)),
            pl.BlockSpec((1, E), lambda i: (0, 0)),
        ],
        out_specs=[
            pl.BlockSpec((GBT, K), lambda i: (i, 0)),
            pl.BlockSpec((GBT, K), lambda i: (i, 0)),
        ],
        out_shape=[
            jax.ShapeDtypeStruct((T, K), jnp.int32),
            jax.ShapeDtypeStruct((T, K), jnp.float32),
        ],
    )(xf, Wg, bg.reshape(1, E))


def _sc_row_gather(table, idx, n_rows, chunk):
    per_w = n_rows // NW
    n_chunks = per_w // chunk
    mesh = plsc.VectorSubcoreMesh(core_axis_name="c", subcore_axis_name="s")

    @functools.partial(
        pl.kernel, mesh=mesh,
        out_type=jax.ShapeDtypeStruct((n_rows, D), jnp.float32),
        scratch_types=[
            pltpu.VMEM((per_w,), jnp.int32),
            pltpu.VMEM((chunk, D), jnp.float32),
            pltpu.VMEM((chunk, D), jnp.float32),
            pltpu.SemaphoreType.DMA,
            pltpu.SemaphoreType.DMA,
            pltpu.SemaphoreType.DMA,
            pltpu.SemaphoreType.DMA,
        ],
    )
    def k(table_hbm, idx_hbm, out_hbm, idx_v, buf0, buf1, g0, g1, w0, w1):
        wid = lax.axis_index("s") * NC + lax.axis_index("c")
        base = wid * per_w
        pltpu.sync_copy(idx_hbm.at[pl.ds(base, per_w)], idx_v)
        bufs, gsem, wsem = [buf0, buf1], [g0, g1], [w0, w1]
        gh = [None] * n_chunks
        wh = [None] * n_chunks
        for c in range(n_chunks):
            if c >= 2:
                wh[c - 2].wait()
            gh[c] = pltpu.async_copy(
                table_hbm.at[idx_v.at[pl.ds(c * chunk, chunk)]],
                bufs[c % 2], gsem[c % 2])
            if c >= 1:
                gh[c - 1].wait()
                wh[c - 1] = pltpu.async_copy(
                    bufs[(c - 1) % 2],
                    out_hbm.at[pl.ds(base + (c - 1) * chunk, chunk)],
                    wsem[(c - 1) % 2])
        gh[n_chunks - 1].wait()
        wh[n_chunks - 1] = pltpu.async_copy(
            bufs[(n_chunks - 1) % 2],
            out_hbm.at[pl.ds(base + (n_chunks - 1) * chunk, chunk)],
            wsem[(n_chunks - 1) % 2])
        wh[n_chunks - 1].wait()
        if n_chunks >= 2:
            wh[n_chunks - 2].wait()

    return k(table, idx)


def _sc_pair_combine(y_sorted, inv):
    per_w = T // NW
    ct = 16
    n_chunks = per_w // ct
    mesh = plsc.VectorSubcoreMesh(core_axis_name="c", subcore_axis_name="s")
    nv = D // NL

    @functools.partial(
        pl.kernel, mesh=mesh,
        out_type=jax.ShapeDtypeStruct((T, D), jnp.float32),
        scratch_types=[
            pltpu.VMEM((2 * per_w,), jnp.int32),
            pltpu.VMEM((2 * ct, D), jnp.float32),
            pltpu.VMEM((2 * ct, D), jnp.float32),
            pltpu.VMEM((ct, D), jnp.float32),
            pltpu.VMEM((ct, D), jnp.float32),
            pltpu.SemaphoreType.DMA,
            pltpu.SemaphoreType.DMA,
            pltpu.SemaphoreType.DMA,
            pltpu.SemaphoreType.DMA,
        ],
    )
    def k(y_hbm, inv_hbm, out_hbm, idx_v, ga, gb, oa, ob, g0, g1, w0, w1):
        wid = lax.axis_index("s") * NC + lax.axis_index("c")
        base = wid * per_w
        pltpu.sync_copy(inv_hbm.at[pl.ds(2 * base, 2 * per_w)], idx_v)
        gbuf, obuf, gsem, wsem = [ga, gb], [oa, ob], [g0, g1], [w0, w1]
        gh = [None] * n_chunks
        wh = [None] * n_chunks
        gh[0] = pltpu.async_copy(
            y_hbm.at[idx_v.at[pl.ds(0, 2 * ct)]], gbuf[0], gsem[0])
        for c in range(n_chunks):
            if c + 1 < n_chunks:
                gh[c + 1] = pltpu.async_copy(
                    y_hbm.at[idx_v.at[pl.ds((c + 1) * 2 * ct, 2 * ct)]],
                    gbuf[(c + 1) % 2], gsem[(c + 1) % 2])
            gh[c].wait()
            if c >= 2:
                wh[c - 2].wait()
            g_v = gbuf[c % 2]
            o_v = obuf[c % 2]

            def add_row(r, _):
                def add_vec(v, _):
                    sl = pl.ds(v * NL, NL)
                    o_v[r, sl] = g_v[2 * r, sl] + g_v[2 * r + 1, sl]
                    return ()
                lax.fori_loop(0, nv, add_vec, (), unroll=8)
                return ()

            lax.fori_loop(0, ct, add_row, ())
            wh[c] = pltpu.async_copy(
                o_v, out_hbm.at[pl.ds(base + c * ct, ct)], wsem[c % 2])
        wh[n_chunks - 1].wait()
        if n_chunks >= 2:
            wh[n_chunks - 2].wait()

    return k(y_sorted, inv)


def _ffn_body(be_ref, x_ref, Wa_ref, ba_ref, W1_ref, b1_ref, W2_ref, b2_ref,
              wgt_ref, tok_ref, y_ref):
    del tok_ref
    xb = x_ref[...]
    ha = jnp.dot(xb, Wa_ref[0], preferred_element_type=jnp.float32) + ba_ref[0]
    ha = ha * jax.nn.sigmoid(ha)
    h1 = jnp.dot(xb, W1_ref[0], preferred_element_type=jnp.float32) + b1_ref[0]
    h = ha * h1
    y = jnp.dot(h, W2_ref[0], preferred_element_type=jnp.float32) + b2_ref[0]
    y_ref[...] = y * wgt_ref[...]


def _grouped_ffn(x_sorted, wgt_sorted, blk_expert, tok_sorted,
                 Wa, ba, W1, b1, W2, b2):
    grid_spec = pltpu.PrefetchScalarGridSpec(
        num_scalar_prefetch=1,
        grid=(NB,),
        in_specs=[
            pl.BlockSpec((BT, D), lambda j, be: (j, 0)),
            pl.BlockSpec((1, D, H), lambda j, be: (be[j], 0, 0)),
            pl.BlockSpec((1, 1, H), lambda j, be: (be[j], 0, 0)),
            pl.BlockSpec((1, D, H), lambda j, be: (be[j], 0, 0)),
            pl.BlockSpec((1, 1, H), lambda j, be: (be[j], 0, 0)),
            pl.BlockSpec((1, H, D), lambda j, be: (be[j], 0, 0)),
            pl.BlockSpec((1, 1, D), lambda j, be: (be[j], 0, 0)),
            pl.BlockSpec((BT, 1), lambda j, be: (j, 0)),
            pl.BlockSpec((1, PT), lambda j, be: (0, 0)),
        ],
        out_specs=pl.BlockSpec((BT, D), lambda j, be: (j, 0)),
    )
    return pl.pallas_call(
        _ffn_body,
        grid_spec=grid_spec,
        out_shape=jax.ShapeDtypeStruct((PT, D), jnp.float32),
        compiler_params=pltpu.CompilerParams(
            dimension_semantics=("arbitrary",),
        ),
    )(blk_expert, x_sorted, Wa, ba.reshape(E, 1, H), W1, b1.reshape(E, 1, H),
      W2, b2.reshape(E, 1, D), wgt_sorted, tok_sorted.reshape(1, PT))


@jax.jit
def kernel(x, Wg, bg, Wa, ba, W1, b1, W2, b2):
    B, S, _ = x.shape
    xf = x.reshape(T, D)

    sel, wgt = _gating(xf, Wg, bg)

    e_flat = sel.reshape(TK)
    w_flat = wgt.reshape(TK)
    order = jnp.argsort(e_flat, stable=True).astype(jnp.int32)
    sizes = jnp.bincount(e_flat, length=E).astype(jnp.int32)
    starts = jnp.cumsum(sizes) - sizes
    padded_sizes = ((sizes + BT - 1) // BT) * BT
    padded_starts = (jnp.cumsum(padded_sizes) - padded_sizes).astype(jnp.int32)
    offs = jnp.arange(NB, dtype=jnp.int32) * BT
    blk_expert = (jnp.sum(padded_starts[None, :] <= offs[:, None],
                          axis=1).astype(jnp.int32) - 1)
    pe = jnp.repeat(blk_expert, BT)
    r = jnp.arange(PT, dtype=jnp.int32) - padded_starts[pe]
    src = starts[pe] + r
    valid = r < sizes[pe]
    g = order[jnp.minimum(src, TK - 1)]
    tok_sorted = jnp.where(valid, g // K, 0)

    x_sorted = _sc_row_gather(xf, tok_sorted, PT, 40)

    wgt_sorted = jnp.where(valid, w_flat[g], 0.0)
    g_sorted = e_flat[order]
    ppos = (padded_starts[g_sorted] + jnp.arange(TK, dtype=jnp.int32)
            - starts[g_sorted])
    inv = ppos[jnp.argsort(order).astype(jnp.int32)]

    y_sorted = _grouped_ffn(x_sorted, wgt_sorted.reshape(PT, 1), blk_expert,
                            tok_sorted, Wa, ba, W1, b1, W2, b2)
    out = _sc_pair_combine(y_sorted, inv)
    return out.reshape(B, S, D)


# speedup vs baseline: 1.0900x; 1.0900x over previous
import functools

import jax
import jax.numpy as jnp
from jax.experimental import pallas as pl
from jax.experimental.pallas import tpu as pltpu

E = 8
K = 2
D = 768
H = 1536

BT = 512
HC = 768


def _moe_body(x_ref, x16_ref, Wg_ref, bg_ref, Wa_ref, ba_ref, W1_ref, b1_ref,
              W2_ref, b2_ref, out_ref, coef_ref):
    e = pl.program_id(1)
    hc = pl.program_id(2)

    @pl.when(jnp.logical_and(e == 0, hc == 0))
    def _gating():
        xb = x_ref[...]
        logits = jnp.dot(xb, Wg_ref[...],
                         preferred_element_type=jnp.float32) + bg_ref[...]
        idx = jax.lax.broadcasted_iota(jnp.int32, logits.shape, 1)
        m1 = jnp.max(logits, axis=-1, keepdims=True)
        a1 = jnp.min(jnp.where(logits == m1, idx, E), axis=-1, keepdims=True)
        l2 = jnp.where(idx == a1, -jnp.inf, logits)
        m2 = jnp.max(l2, axis=-1, keepdims=True)
        a2 = jnp.min(jnp.where(l2 == m2, idx, E), axis=-1, keepdims=True)
        e2 = jnp.exp(m2 - m1)
        s = 1.0 + e2
        w1 = 1.0 / s
        w2 = e2 / s
        coef_ref[...] = jnp.where(idx == a1, w1,
                                  jnp.where(idx == a2, w2, 0.0))
        out_ref[...] = jnp.zeros_like(out_ref)

    xb = x16_ref[...]
    ha = jnp.dot(xb, Wa_ref[0],
                 preferred_element_type=jnp.float32) + ba_ref[0]
    ha = ha * jax.nn.sigmoid(ha)
    h1 = jnp.dot(xb, W1_ref[0],
                 preferred_element_type=jnp.float32) + b1_ref[0]
    h = ha * h1
    y = jnp.dot(h.astype(jnp.bfloat16), W2_ref[0],
                preferred_element_type=jnp.float32)

    idx = jax.lax.broadcasted_iota(jnp.int32, coef_ref.shape, 1)
    c = jnp.sum(coef_ref[...] * (idx == e), axis=-1, keepdims=True)

    @pl.when(hc == 0)
    def _add_bias():
        out_ref[...] += c * (y + b2_ref[0])

    @pl.when(hc != 0)
    def _no_bias():
        out_ref[...] += c * y


@functools.partial(jax.jit, static_argnames=())
def kernel(x, Wg, bg, Wa, ba, W1, b1, W2, b2):
    B, S, _ = x.shape
    T = B * S
    xf = x.reshape(T, D)
    grid = (T // BT, E, H // HC)
    out = pl.pallas_call(
        _moe_body,
        grid=grid,
        in_specs=[
            pl.BlockSpec((BT, D), lambda i, e, h: (i, 0)),
            pl.BlockSpec((BT, D), lambda i, e, h: (i, 0)),
            pl.BlockSpec((D, E), lambda i, e, h: (0, 0)),
            pl.BlockSpec((1, E), lambda i, e, h: (0, 0)),
            pl.BlockSpec((1, D, HC), lambda i, e, h: (e, 0, h)),
            pl.BlockSpec((1, 1, HC), lambda i, e, h: (e, 0, h)),
            pl.BlockSpec((1, D, HC), lambda i, e, h: (e, 0, h)),
            pl.BlockSpec((1, 1, HC), lambda i, e, h: (e, 0, h)),
            pl.BlockSpec((1, HC, D), lambda i, e, h: (e, h, 0)),
            pl.BlockSpec((1, 1, D), lambda i, e, h: (e, 0, 0)),
        ],
        out_specs=pl.BlockSpec((BT, D), lambda i, e, h: (i, 0)),
        out_shape=jax.ShapeDtypeStruct((T, D), jnp.float32),
        scratch_shapes=[pltpu.VMEM((BT, E), jnp.float32)],
        compiler_params=pltpu.CompilerParams(
            dimension_semantics=("parallel", "arbitrary", "arbitrary"),
        ),
    )(xf, xf.astype(jnp.bfloat16), Wg, bg.reshape(1, E),
      Wa.astype(jnp.bfloat16), ba.reshape(E, 1, H),
      W1.astype(jnp.bfloat16), b1.reshape(E, 1, H),
      W2.astype(jnp.bfloat16), b2.reshape(E, 1, D))
    return out.reshape(B, S, D)


# speedup vs baseline: 1.3565x; 1.2445x over previous
import functools

import jax
import jax.numpy as jnp
from jax import lax
from jax.experimental import pallas as pl
from jax.experimental.pallas import tpu as pltpu
from jax.experimental.pallas import tpu_sc as plsc

E = 8
K = 2
D = 768
H = 1536
T = 4096
TK = T * K
BT = 256
PT = TK + E * BT
NB = PT // BT

GBT = 512

NC = 2
NS = 16
NW = NC * NS
NL = 16


def _gating_body(x_ref, Wg_ref, bg_ref, sel_ref, wgt_ref):
    logits = jnp.dot(x_ref[...], Wg_ref[...],
                     preferred_element_type=jnp.float32) + bg_ref[...]
    idx = jax.lax.broadcasted_iota(jnp.int32, logits.shape, 1)
    m1 = jnp.max(logits, axis=-1, keepdims=True)
    a1 = jnp.min(jnp.where(logits == m1, idx, E), axis=-1, keepdims=True)
    l2 = jnp.where(idx == a1, -jnp.inf, logits)
    m2 = jnp.max(l2, axis=-1, keepdims=True)
    a2 = jnp.min(jnp.where(l2 == m2, idx, E), axis=-1, keepdims=True)
    e2 = jnp.exp(m2 - m1)
    s = 1.0 + e2
    sel_ref[...] = jnp.concatenate([a1, a2], axis=1)
    wgt_ref[...] = jnp.concatenate([1.0 / s, e2 / s], axis=1)


def _gating(xf, Wg, bg):
    return pl.pallas_call(
        _gating_body,
        grid=(T // GBT,),
        in_specs=[
            pl.BlockSpec((GBT, D), lambda i: (i, 0)),
            pl.BlockSpec((D, E), lambda i: (0, 0)),
            pl.BlockSpec((1, E), lambda i: (0, 0)),
        ],
        out_specs=[
            pl.BlockSpec((GBT, K), lambda i: (i, 0)),
            pl.BlockSpec((GBT, K), lambda i: (i, 0)),
        ],
        out_shape=[
            jax.ShapeDtypeStruct((T, K), jnp.int32),
            jax.ShapeDtypeStruct((T, K), jnp.float32),
        ],
    )(xf, Wg, bg.reshape(1, E))


def _sc_row_gather(table, idx, n_rows, chunk):
    per_w = n_rows // NW
    n_chunks = per_w // chunk
    mesh = plsc.VectorSubcoreMesh(core_axis_name="c", subcore_axis_name="s")

    @functools.partial(
        pl.kernel, mesh=mesh,
        out_type=jax.ShapeDtypeStruct((n_rows, D), jnp.float32),
        scratch_types=[
            pltpu.VMEM((per_w,), jnp.int32),
            pltpu.VMEM((chunk, D), jnp.float32),
            pltpu.VMEM((chunk, D), jnp.float32),
            pltpu.SemaphoreType.DMA,
            pltpu.SemaphoreType.DMA,
            pltpu.SemaphoreType.DMA,
            pltpu.SemaphoreType.DMA,
        ],
    )
    def k(table_hbm, idx_hbm, out_hbm, idx_v, buf0, buf1, g0, g1, w0, w1):
        wid = lax.axis_index("s") * NC + lax.axis_index("c")
        base = wid * per_w
        pltpu.sync_copy(idx_hbm.at[pl.ds(base, per_w)], idx_v)
        bufs, gsem, wsem = [buf0, buf1], [g0, g1], [w0, w1]
        gh = [None] * n_chunks
        wh = [None] * n_chunks
        for c in range(n_chunks):
            if c >= 2:
                wh[c - 2].wait()
            gh[c] = pltpu.async_copy(
                table_hbm.at[idx_v.at[pl.ds(c * chunk, chunk)]],
                bufs[c % 2], gsem[c % 2])
            if c >= 1:
                gh[c - 1].wait()
                wh[c - 1] = pltpu.async_copy(
                    bufs[(c - 1) % 2],
                    out_hbm.at[pl.ds(base + (c - 1) * chunk, chunk)],
                    wsem[(c - 1) % 2])
        gh[n_chunks - 1].wait()
        wh[n_chunks - 1] = pltpu.async_copy(
            bufs[(n_chunks - 1) % 2],
            out_hbm.at[pl.ds(base + (n_chunks - 1) * chunk, chunk)],
            wsem[(n_chunks - 1) % 2])
        wh[n_chunks - 1].wait()
        if n_chunks >= 2:
            wh[n_chunks - 2].wait()

    return k(table, idx)


def _sc_row_gather_spmem(table, idx, n_rows, chunk):
    NP = 6
    D2 = D // NP
    RS = T // NS
    per_t = n_rows // NS
    n_chunks = per_t // chunk
    mesh = plsc.VectorSubcoreMesh(core_axis_name="c", subcore_axis_name="s")

    @functools.partial(
        pl.kernel, mesh=mesh,
        out_type=jax.ShapeDtypeStruct((n_rows, D), jnp.float32),
        scratch_types=[
            pltpu.VMEM((per_t,), jnp.int32),
            pltpu.VMEM((chunk, D2), jnp.float32),
            pltpu.VMEM((chunk, D2), jnp.float32),
            pltpu.VMEM_SHARED((T, D2), jnp.float32),
            pltpu.SemaphoreType.DMA,
            pltpu.SemaphoreType.DMA,
            pltpu.SemaphoreType.DMA,
            pltpu.SemaphoreType.DMA,
        ],
    )
    def k(table_hbm, idx_hbm, out_hbm, idx_v, buf0, buf1, shared,
          g0, g1, w0, w1):
        cid = lax.axis_index("c")
        sid = lax.axis_index("s")
        pltpu.sync_copy(idx_hbm.at[pl.ds(sid * per_t, per_t)], idx_v)
        bufs, gsem, wsem = [buf0, buf1], [g0, g1], [w0, w1]
        for p in range(NP // NC):
            col0 = (p * NC + 0) * D2
            colv = col0 + cid * D2
            pltpu.sync_copy(
                table_hbm.at[pl.ds(sid * RS, RS), pl.ds(colv, D2)],
                shared.at[pl.ds(sid * RS, RS)])
            plsc.subcore_barrier()
            gh = [None] * n_chunks
            wh = [None] * n_chunks
            for c in range(n_chunks):
                if c >= 2:
                    wh[c - 2].wait()
                gh[c] = pltpu.async_copy(
                    shared.at[idx_v.at[pl.ds(c * chunk, chunk)]],
                    bufs[c % 2], gsem[c % 2])
                if c >= 1:
                    gh[c - 1].wait()
                    wh[c - 1] = pltpu.async_copy(
                        bufs[(c - 1) % 2],
                        out_hbm.at[pl.ds(sid * per_t + (c - 1) * chunk, chunk),
                                   pl.ds(colv, D2)],
                        wsem[(c - 1) % 2])
            gh[n_chunks - 1].wait()
            wh[n_chunks - 1] = pltpu.async_copy(
                bufs[(n_chunks - 1) % 2],
                out_hbm.at[pl.ds(sid * per_t + (n_chunks - 1) * chunk, chunk),
                           pl.ds(colv, D2)],
                wsem[(n_chunks - 1) % 2])
            wh[n_chunks - 1].wait()
            if n_chunks >= 2:
                wh[n_chunks - 2].wait()
            plsc.subcore_barrier()

    return k(table, idx)


def _sc_pair_combine(y_sorted, inv):
    per_w = T // NW
    ct = 16
    n_chunks = per_w // ct
    mesh = plsc.VectorSubcoreMesh(core_axis_name="c", subcore_axis_name="s")
    nv = D // NL

    @functools.partial(
        pl.kernel, mesh=mesh,
        out_type=jax.ShapeDtypeStruct((T, D), jnp.float32),
        scratch_types=[
            pltpu.VMEM((2 * per_w,), jnp.int32),
            pltpu.VMEM((2 * ct, D), jnp.float32),
            pltpu.VMEM((2 * ct, D), jnp.float32),
            pltpu.VMEM((ct, D), jnp.float32),
            pltpu.VMEM((ct, D), jnp.float32),
            pltpu.SemaphoreType.DMA,
            pltpu.SemaphoreType.DMA,
            pltpu.SemaphoreType.DMA,
            pltpu.SemaphoreType.DMA,
        ],
    )
    def k(y_hbm, inv_hbm, out_hbm, idx_v, ga, gb, oa, ob, g0, g1, w0, w1):
        wid = lax.axis_index("s") * NC + lax.axis_index("c")
        base = wid * per_w
        pltpu.sync_copy(inv_hbm.at[pl.ds(2 * base, 2 * per_w)], idx_v)
        gbuf, obuf, gsem, wsem = [ga, gb], [oa, ob], [g0, g1], [w0, w1]
        gh = [None] * n_chunks
        wh = [None] * n_chunks
        gh[0] = pltpu.async_copy(
            y_hbm.at[idx_v.at[pl.ds(0, 2 * ct)]], gbuf[0], gsem[0])
        for c in range(n_chunks):
            if c + 1 < n_chunks:
                gh[c + 1] = pltpu.async_copy(
                    y_hbm.at[idx_v.at[pl.ds((c + 1) * 2 * ct, 2 * ct)]],
                    gbuf[(c + 1) % 2], gsem[(c + 1) % 2])
            gh[c].wait()
            if c >= 2:
                wh[c - 2].wait()
            g_v = gbuf[c % 2]
            o_v = obuf[c % 2]

            def add_row(r, _):
                def add_vec(v, _):
                    sl = pl.ds(v * NL, NL)
                    o_v[r, sl] = g_v[2 * r, sl] + g_v[2 * r + 1, sl]
                    return ()
                lax.fori_loop(0, nv, add_vec, (), unroll=8)
                return ()

            lax.fori_loop(0, ct, add_row, ())
            wh[c] = pltpu.async_copy(
                o_v, out_hbm.at[pl.ds(base + c * ct, ct)], wsem[c % 2])
        wh[n_chunks - 1].wait()
        if n_chunks >= 2:
            wh[n_chunks - 2].wait()

    return k(y_sorted, inv)


def _ffn_body(be_ref, x_ref, Wa_ref, ba_ref, W1_ref, b1_ref, W2_ref, b2_ref,
              wgt_ref, tok_ref, y_ref):
    del tok_ref
    xb = x_ref[...]
    ha = jnp.dot(xb, Wa_ref[0], preferred_element_type=jnp.float32) + ba_ref[0]
    ha = ha * jax.nn.sigmoid(ha)
    h1 = jnp.dot(xb, W1_ref[0], preferred_element_type=jnp.float32) + b1_ref[0]
    h = ha * h1
    y = jnp.dot(h, W2_ref[0], preferred_element_type=jnp.float32) + b2_ref[0]
    y_ref[...] = y * wgt_ref[...]


def _grouped_ffn(x_sorted, wgt_sorted, blk_expert, tok_sorted,
                 Wa, ba, W1, b1, W2, b2):
    grid_spec = pltpu.PrefetchScalarGridSpec(
        num_scalar_prefetch=1,
        grid=(NB,),
        in_specs=[
            pl.BlockSpec((BT, D), lambda j, be: (j, 0)),
            pl.BlockSpec((1, D, H), lambda j, be: (be[j], 0, 0)),
            pl.BlockSpec((1, 1, H), lambda j, be: (be[j], 0, 0)),
            pl.BlockSpec((1, D, H), lambda j, be: (be[j], 0, 0)),
            pl.BlockSpec((1, 1, H), lambda j, be: (be[j], 0, 0)),
            pl.BlockSpec((1, H, D), lambda j, be: (be[j], 0, 0)),
            pl.BlockSpec((1, 1, D), lambda j, be: (be[j], 0, 0)),
            pl.BlockSpec((BT, 1), lambda j, be: (j, 0)),
            pl.BlockSpec((1, PT), lambda j, be: (0, 0)),
        ],
        out_specs=pl.BlockSpec((BT, D), lambda j, be: (j, 0)),
    )
    return pl.pallas_call(
        _ffn_body,
        grid_spec=grid_spec,
        out_shape=jax.ShapeDtypeStruct((PT, D), jnp.float32),
        compiler_params=pltpu.CompilerParams(
            dimension_semantics=("arbitrary",),
        ),
    )(blk_expert, x_sorted, Wa, ba.reshape(E, 1, H), W1, b1.reshape(E, 1, H),
      W2, b2.reshape(E, 1, D), wgt_sorted, tok_sorted.reshape(1, PT))


@jax.jit
def kernel(x, Wg, bg, Wa, ba, W1, b1, W2, b2):
    B, S, _ = x.shape
    xf = x.reshape(T, D)

    sel, wgt = _gating(xf, Wg, bg)

    e_flat = sel.reshape(TK)
    w_flat = wgt.reshape(TK)
    order = jnp.argsort(e_flat, stable=True).astype(jnp.int32)
    sizes = jnp.bincount(e_flat, length=E).astype(jnp.int32)
    starts = jnp.cumsum(sizes) - sizes
    padded_sizes = ((sizes + BT - 1) // BT) * BT
    padded_starts = (jnp.cumsum(padded_sizes) - padded_sizes).astype(jnp.int32)
    offs = jnp.arange(NB, dtype=jnp.int32) * BT
    blk_expert = (jnp.sum(padded_starts[None, :] <= offs[:, None],
                          axis=1).astype(jnp.int32) - 1)
    pe = jnp.repeat(blk_expert, BT)
    r = jnp.arange(PT, dtype=jnp.int32) - padded_starts[pe]
    src = starts[pe] + r
    valid = r < sizes[pe]
    g = order[jnp.minimum(src, TK - 1)]
    tok_sorted = jnp.where(valid, g // K, 0)

    x_sorted = _sc_row_gather_spmem(xf, tok_sorted, PT, 128)

    wgt_sorted = jnp.where(valid, w_flat[g], 0.0)
    g_sorted = e_flat[order]
    ppos = (padded_starts[g_sorted] + jnp.arange(TK, dtype=jnp.int32)
            - starts[g_sorted])
    inv = ppos[jnp.argsort(order).astype(jnp.int32)]

    y_sorted = _grouped_ffn(x_sorted, wgt_sorted.reshape(PT, 1), blk_expert,
                            tok_sorted, Wa, ba, W1, b1, W2, b2)
    out = _sc_pair_combine(y_sorted, inv)
    return out.reshape(B, S, D)


# speedup vs baseline: 1.3836x; 1.0199x over previous
import functools

import jax
import jax.numpy as jnp
from jax import lax
from jax.experimental import pallas as pl
from jax.experimental.pallas import tpu as pltpu
from jax.experimental.pallas import tpu_sc as plsc

E = 8
K = 2
D = 768
H = 1536
T = 4096
TK = T * K
BT = 256
PT = TK + E * BT
NB = PT // BT

GBT = 512

NC = 2
NS = 16
NW = NC * NS
NL = 16


def _gating_body(x_ref, Wg_ref, bg_ref, sel_ref, wgt_ref, cnt_ref):
    i = pl.program_id(0)
    logits = jnp.dot(x_ref[...], Wg_ref[...],
                     preferred_element_type=jnp.float32) + bg_ref[...]
    idx = jax.lax.broadcasted_iota(jnp.int32, logits.shape, 1)
    m1 = jnp.max(logits, axis=-1, keepdims=True)
    a1 = jnp.min(jnp.where(logits == m1, idx, E), axis=-1, keepdims=True)
    l2 = jnp.where(idx == a1, -jnp.inf, logits)
    m2 = jnp.max(l2, axis=-1, keepdims=True)
    a2 = jnp.min(jnp.where(l2 == m2, idx, E), axis=-1, keepdims=True)
    e2 = jnp.exp(m2 - m1)
    s = 1.0 + e2
    sel_ref[...] = jnp.concatenate([a1, a2], axis=1)
    wgt_ref[...] = jnp.concatenate([1.0 / s, e2 / s], axis=1)

    @pl.when(i == 0)
    def _init():
        cnt_ref[...] = jnp.zeros_like(cnt_ref)

    hits = ((a1 == idx).astype(jnp.int32) + (a2 == idx).astype(jnp.int32))
    cnt_ref[...] += jnp.sum(hits, axis=0, keepdims=True)


def _gating(xf, Wg, bg):
    return pl.pallas_call(
        _gating_body,
        grid=(T // GBT,),
        in_specs=[
            pl.BlockSpec((GBT, D), lambda i: (i, 0)),
            pl.BlockSpec((D, E), lambda i: (0, 0)),
            pl.BlockSpec((1, E), lambda i: (0, 0)),
        ],
        out_specs=[
            pl.BlockSpec((GBT, K), lambda i: (i, 0)),
            pl.BlockSpec((GBT, K), lambda i: (i, 0)),
            pl.BlockSpec((1, E), lambda i: (0, 0)),
        ],
        out_shape=[
            jax.ShapeDtypeStruct((T, K), jnp.int32),
            jax.ShapeDtypeStruct((T, K), jnp.float32),
            jax.ShapeDtypeStruct((1, E), jnp.int32),
        ],
    )(xf, Wg, bg.reshape(1, E))


def _sc_row_gather(table, idx, n_rows, chunk):
    per_w = n_rows // NW
    n_chunks = per_w // chunk
    mesh = plsc.VectorSubcoreMesh(core_axis_name="c", subcore_axis_name="s")

    @functools.partial(
        pl.kernel, mesh=mesh,
        out_type=jax.ShapeDtypeStruct((n_rows, D), jnp.float32),
        scratch_types=[
            pltpu.VMEM((per_w,), jnp.int32),
            pltpu.VMEM((chunk, D), jnp.float32),
            pltpu.VMEM((chunk, D), jnp.float32),
            pltpu.SemaphoreType.DMA,
            pltpu.SemaphoreType.DMA,
            pltpu.SemaphoreType.DMA,
            pltpu.SemaphoreType.DMA,
        ],
    )
    def k(table_hbm, idx_hbm, out_hbm, idx_v, buf0, buf1, g0, g1, w0, w1):
        wid = lax.axis_index("s") * NC + lax.axis_index("c")
        base = wid * per_w
        pltpu.sync_copy(idx_hbm.at[pl.ds(base, per_w)], idx_v)
        bufs, gsem, wsem = [buf0, buf1], [g0, g1], [w0, w1]
        gh = [None] * n_chunks
        wh = [None] * n_chunks
        for c in range(n_chunks):
            if c >= 2:
                wh[c - 2].wait()
            gh[c] = pltpu.async_copy(
                table_hbm.at[idx_v.at[pl.ds(c * chunk, chunk)]],
                bufs[c % 2], gsem[c % 2])
            if c >= 1:
                gh[c - 1].wait()
                wh[c - 1] = pltpu.async_copy(
                    bufs[(c - 1) % 2],
                    out_hbm.at[pl.ds(base + (c - 1) * chunk, chunk)],
                    wsem[(c - 1) % 2])
        gh[n_chunks - 1].wait()
        wh[n_chunks - 1] = pltpu.async_copy(
            bufs[(n_chunks - 1) % 2],
            out_hbm.at[pl.ds(base + (n_chunks - 1) * chunk, chunk)],
            wsem[(n_chunks - 1) % 2])
        wh[n_chunks - 1].wait()
        if n_chunks >= 2:
            wh[n_chunks - 2].wait()

    return k(table, idx)


def _sc_row_gather_spmem(table, idx, n_rows, chunk):
    NP = 6
    D2 = D // NP
    RS = T // NS
    per_t = n_rows // NS
    n_chunks = per_t // chunk
    mesh = plsc.VectorSubcoreMesh(core_axis_name="c", subcore_axis_name="s")

    @functools.partial(
        pl.kernel, mesh=mesh,
        out_type=jax.ShapeDtypeStruct((n_rows, D), jnp.float32),
        scratch_types=[
            pltpu.VMEM((per_t,), jnp.int32),
            pltpu.VMEM((chunk, D2), jnp.float32),
            pltpu.VMEM((chunk, D2), jnp.float32),
            pltpu.VMEM_SHARED((T, D2), jnp.float32),
            pltpu.SemaphoreType.DMA,
            pltpu.SemaphoreType.DMA,
            pltpu.SemaphoreType.DMA,
            pltpu.SemaphoreType.DMA,
        ],
    )
    def k(table_hbm, idx_hbm, out_hbm, idx_v, buf0, buf1, shared,
          g0, g1, w0, w1):
        cid = lax.axis_index("c")
        sid = lax.axis_index("s")
        pltpu.sync_copy(idx_hbm.at[pl.ds(sid * per_t, per_t)], idx_v)
        bufs, gsem, wsem = [buf0, buf1], [g0, g1], [w0, w1]
        for p in range(NP // NC):
            col0 = (p * NC + 0) * D2
            colv = col0 + cid * D2
            pltpu.sync_copy(
                table_hbm.at[pl.ds(sid * RS, RS), pl.ds(colv, D2)],
                shared.at[pl.ds(sid * RS, RS)])
            plsc.subcore_barrier()
            gh = [None] * n_chunks
            wh = [None] * n_chunks
            for c in range(n_chunks):
                if c >= 2:
                    wh[c - 2].wait()
                gh[c] = pltpu.async_copy(
                    shared.at[idx_v.at[pl.ds(c * chunk, chunk)]],
                    bufs[c % 2], gsem[c % 2])
                if c >= 1:
                    gh[c - 1].wait()
                    wh[c - 1] = pltpu.async_copy(
                        bufs[(c - 1) % 2],
                        out_hbm.at[pl.ds(sid * per_t + (c - 1) * chunk, chunk),
                                   pl.ds(colv, D2)],
                        wsem[(c - 1) % 2])
            gh[n_chunks - 1].wait()
            wh[n_chunks - 1] = pltpu.async_copy(
                bufs[(n_chunks - 1) % 2],
                out_hbm.at[pl.ds(sid * per_t + (n_chunks - 1) * chunk, chunk),
                           pl.ds(colv, D2)],
                wsem[(n_chunks - 1) % 2])
            wh[n_chunks - 1].wait()
            if n_chunks >= 2:
                wh[n_chunks - 2].wait()
            plsc.subcore_barrier()

    return k(table, idx)


def _sc_pair_combine(y_sorted, inv):
    per_w = T // NW
    ct = 16
    n_chunks = per_w // ct
    mesh = plsc.VectorSubcoreMesh(core_axis_name="c", subcore_axis_name="s")
    nv = D // NL

    @functools.partial(
        pl.kernel, mesh=mesh,
        out_type=jax.ShapeDtypeStruct((T, D), jnp.float32),
        scratch_types=[
            pltpu.VMEM((2 * per_w,), jnp.int32),
            pltpu.VMEM((2 * ct, D), jnp.float32),
            pltpu.VMEM((2 * ct, D), jnp.float32),
            pltpu.VMEM((ct, D), jnp.float32),
            pltpu.VMEM((ct, D), jnp.float32),
            pltpu.SemaphoreType.DMA,
            pltpu.SemaphoreType.DMA,
            pltpu.SemaphoreType.DMA,
            pltpu.SemaphoreType.DMA,
        ],
    )
    def k(y_hbm, inv_hbm, out_hbm, idx_v, ga, gb, oa, ob, g0, g1, w0, w1):
        wid = lax.axis_index("s") * NC + lax.axis_index("c")
        base = wid * per_w
        pltpu.sync_copy(inv_hbm.at[pl.ds(2 * base, 2 * per_w)], idx_v)
        gbuf, obuf, gsem, wsem = [ga, gb], [oa, ob], [g0, g1], [w0, w1]
        gh = [None] * n_chunks
        wh = [None] * n_chunks
        gh[0] = pltpu.async_copy(
            y_hbm.at[idx_v.at[pl.ds(0, 2 * ct)]], gbuf[0], gsem[0])
        for c in range(n_chunks):
            if c + 1 < n_chunks:
                gh[c + 1] = pltpu.async_copy(
                    y_hbm.at[idx_v.at[pl.ds((c + 1) * 2 * ct, 2 * ct)]],
                    gbuf[(c + 1) % 2], gsem[(c + 1) % 2])
            gh[c].wait()
            if c >= 2:
                wh[c - 2].wait()
            g_v = gbuf[c % 2]
            o_v = obuf[c % 2]

            def add_row(r, _):
                def add_vec(v, _):
                    sl = pl.ds(v * NL, NL)
                    o_v[r, sl] = g_v[2 * r, sl] + g_v[2 * r + 1, sl]
                    return ()
                lax.fori_loop(0, nv, add_vec, (), unroll=8)
                return ()

            lax.fori_loop(0, ct, add_row, ())
            wh[c] = pltpu.async_copy(
                o_v, out_hbm.at[pl.ds(base + c * ct, ct)], wsem[c % 2])
        wh[n_chunks - 1].wait()
        if n_chunks >= 2:
            wh[n_chunks - 2].wait()

    return k(y_sorted, inv)


def _ffn_body(be_ref, x_ref, Wa_ref, ba_ref, W1_ref, b1_ref, W2_ref, b2_ref,
              wgt_ref, tok_ref, y_ref):
    del tok_ref
    xb = x_ref[...]
    ha = jnp.dot(xb, Wa_ref[0], preferred_element_type=jnp.float32) + ba_ref[0]
    ha = ha * jax.nn.sigmoid(ha)
    h1 = jnp.dot(xb, W1_ref[0], preferred_element_type=jnp.float32) + b1_ref[0]
    h = ha * h1
    y = jnp.dot(h, W2_ref[0], preferred_element_type=jnp.float32) + b2_ref[0]
    y_ref[...] = y * wgt_ref[...]


def _grouped_ffn(x_sorted, wgt_sorted, blk_expert, tok_sorted,
                 Wa, ba, W1, b1, W2, b2):
    grid_spec = pltpu.PrefetchScalarGridSpec(
        num_scalar_prefetch=1,
        grid=(NB,),
        in_specs=[
            pl.BlockSpec((BT, D), lambda j, be: (j, 0)),
            pl.BlockSpec((1, D, H), lambda j, be: (be[j], 0, 0)),
            pl.BlockSpec((1, 1, H), lambda j, be: (be[j], 0, 0)),
            pl.BlockSpec((1, D, H), lambda j, be: (be[j], 0, 0)),
            pl.BlockSpec((1, 1, H), lambda j, be: (be[j], 0, 0)),
            pl.BlockSpec((1, H, D), lambda j, be: (be[j], 0, 0)),
            pl.BlockSpec((1, 1, D), lambda j, be: (be[j], 0, 0)),
            pl.BlockSpec((BT, 1), lambda j, be: (j, 0)),
            pl.BlockSpec((1, PT), lambda j, be: (0, 0)),
        ],
        out_specs=pl.BlockSpec((BT, D), lambda j, be: (j, 0)),
    )
    return pl.pallas_call(
        _ffn_body,
        grid_spec=grid_spec,
        out_shape=jax.ShapeDtypeStruct((PT, D), jnp.float32),
        compiler_params=pltpu.CompilerParams(
            dimension_semantics=("arbitrary",),
        ),
    )(blk_expert, x_sorted, Wa, ba.reshape(E, 1, H), W1, b1.reshape(E, 1, H),
      W2, b2.reshape(E, 1, D), wgt_sorted, tok_sorted.reshape(1, PT))


@jax.jit
def kernel(x, Wg, bg, Wa, ba, W1, b1, W2, b2):
    B, S, _ = x.shape
    xf = x.reshape(T, D)

    sel, wgt, cnt = _gating(xf, Wg, bg)

    e_flat = sel.reshape(TK)
    w_flat = wgt.reshape(TK)
    order = jnp.argsort(e_flat, stable=True).astype(jnp.int32)
    sizes = cnt.reshape(E)
    starts = jnp.cumsum(sizes) - sizes
    padded_sizes = ((sizes + BT - 1) // BT) * BT
    padded_starts = (jnp.cumsum(padded_sizes) - padded_sizes).astype(jnp.int32)
    offs = jnp.arange(NB, dtype=jnp.int32) * BT
    blk_expert = (jnp.sum(padded_starts[None, :] <= offs[:, None],
                          axis=1).astype(jnp.int32) - 1)
    pe = jnp.repeat(blk_expert, BT)
    r = jnp.arange(PT, dtype=jnp.int32) - padded_starts[pe]
    src = starts[pe] + r
    valid = r < sizes[pe]
    g = order[jnp.minimum(src, TK - 1)]
    tok_sorted = jnp.where(valid, g // K, 0)

    x_sorted = _sc_row_gather_spmem(xf, tok_sorted, PT, 128)

    wgt_sorted = jnp.where(valid, w_flat[g], 0.0)
    g_sorted = e_flat[order]
    ppos = (padded_starts[g_sorted] + jnp.arange(TK, dtype=jnp.int32)
            - starts[g_sorted])
    inv = ppos[jnp.argsort(order).astype(jnp.int32)]

    y_sorted = _grouped_ffn(x_sorted, wgt_sorted.reshape(PT, 1), blk_expert,
                            tok_sorted, Wa, ba, W1, b1, W2, b2)
    out = _sc_pair_combine(y_sorted, inv)
    return out.reshape(B, S, D)
